# baseline (speedup 1.0000x reference)
import jax
import jax.numpy as jnp
from jax import lax
from jax.experimental import pallas as pl
from jax.experimental.pallas import tpu as pltpu
from jax.experimental.pallas import tpu_sc as plsc

_N = 6422528
_NW = 32
_PER_W = _N // _NW
_CH = 28672
_NCHUNK = _PER_W // _CH


def _sc_body(o_hbm, zbuf):
    wid = lax.axis_index("s") * 2 + lax.axis_index("c")
    base = wid * _PER_W

    def _zero(i, carry):
        zbuf[pl.ds(i * 16, 16)] = jnp.zeros((16,), jnp.float32)
        return carry

    lax.fori_loop(0, _CH // 16, _zero, 0)

    def _emit(j, carry):
        pltpu.sync_copy(zbuf, o_hbm.at[pl.ds(base + j * _CH, _CH)])
        return carry

    lax.fori_loop(0, _NCHUNK, _emit, 0)


def kernel(x):
    del x
    mesh = plsc.VectorSubcoreMesh(core_axis_name="c", subcore_axis_name="s")
    fill = pl.kernel(
        _sc_body,
        mesh=mesh,
        out_type=jax.ShapeDtypeStruct((_N,), jnp.float32),
        scratch_types=[pltpu.VMEM((_CH,), jnp.float32)],
    )
    return fill()


# speedup vs baseline: 3.2987x; 3.2987x over previous
import jax
import jax.numpy as jnp
from jax.experimental import pallas as pl

_N = 6422528
_BN = _N // 16


def _fill_body(o_ref):
    o_ref[...] = jnp.zeros((_BN,), jnp.float32)


def kernel(x):
    del x
    return pl.pallas_call(
        _fill_body,
        grid=(16,),
        out_specs=pl.BlockSpec((_BN,), lambda i: (i,)),
        out_shape=jax.ShapeDtypeStruct((_N,), jnp.float32),
    )()


# speedup vs baseline: 3.7301x; 1.1308x over previous
import jax
import jax.numpy as jnp
from jax.experimental import pallas as pl

_N = 6422528
_BN = _N // 4


def _fill_body(o_ref):
    o_ref[...] = jnp.zeros((_BN,), jnp.float32)


def kernel(x):
    del x
    return pl.pallas_call(
        _fill_body,
        grid=(4,),
        out_specs=pl.BlockSpec((_BN,), lambda i: (i,)),
        out_shape=jax.ShapeDtypeStruct((_N,), jnp.float32),
    )()


# speedup vs baseline: 4.0997x; 1.0991x over previous
import jax
import jax.numpy as jnp
from jax.experimental import pallas as pl

_N = 6422528
_BN = _N // 8


def _fill_body(o_ref):
    o_ref[...] = jnp.zeros((_BN,), jnp.float32)


def kernel(x):
    del x
    return pl.pallas_call(
        _fill_body,
        grid=(8,),
        out_specs=pl.BlockSpec((_BN,), lambda i: (i,)),
        out_shape=jax.ShapeDtypeStruct((_N,), jnp.float32),
    )()
